# Initial kernel scaffold; baseline (speedup 1.0000x reference)
import jax
import jax.numpy as jnp
from jax.experimental import pallas as pl


def kernel(query_idxs, doc_idxs, query_table, doc_table, Wq, bq, Wd, bd):
    raise NotImplementedError("write your pallas kernel here")


# speedup vs baseline: 2.5373x; 2.5373x over previous
import functools

import jax
import jax.numpy as jnp
from jax import lax
from jax.experimental import pallas as pl
from jax.experimental.pallas import tpu as pltpu
from jax.experimental.pallas import tpu_sc as plsc

EMB = 64
NC, NS = 2, 16
NW = NC * NS
CH = 128


def _sc_body(L, nch, q_idx3, d_idx3, q_table, d_table, q_out, d_out,
             idx_v, acc, sem):
    wid = lax.axis_index("s") * NC + lax.axis_index("c")
    bpw = nch * CH
    base_ch = wid * nch

    def tower(idx3_hbm, table_hbm, out_hbm):
        pltpu.sync_copy(idx3_hbm.at[:, pl.ds(base_ch, nch), :], idx_v)
        for c in range(nch):
            pltpu.async_copy(
                table_hbm.at[idx_v.at[0, c]],
                acc.at[pl.ds(c * CH, CH), :], sem).wait()

        def step(l, _):
            for c in range(nch):
                pltpu.async_copy(
                    table_hbm.at[idx_v.at[l, c]],
                    acc.at[pl.ds(c * CH, CH), :], sem, add=True).wait()
            return 0

        lax.fori_loop(1, L, step, 0)
        pltpu.sync_copy(acc, out_hbm.at[pl.ds(wid * bpw, bpw), :])

    tower(q_idx3, q_table, q_out)
    tower(d_idx3, d_table, d_out)


def _sc_sum(q_idx3, d_idx3, q_table, d_table):
    L, nblk, _ = q_idx3.shape
    B = nblk * CH
    nch = nblk // NW
    mesh = plsc.VectorSubcoreMesh(
        core_axis_name="c", subcore_axis_name="s",
        num_cores=NC, num_subcores=NS)
    f = functools.partial(
        pl.kernel,
        functools.partial(_sc_body, L, nch),
        out_type=(jax.ShapeDtypeStruct((B, EMB), jnp.float32),
                  jax.ShapeDtypeStruct((B, EMB), jnp.float32)),
        mesh=mesh,
        compiler_params=pltpu.CompilerParams(use_tc_tiling_on_sc=False),
        scratch_types=[
            pltpu.VMEM((L, nch, CH), jnp.int32),
            pltpu.VMEM((nch * CH, EMB), jnp.float32),
            pltpu.SemaphoreType.DMA,
        ],
    )()
    return f(q_idx3, d_idx3, q_table, d_table)


def _tc_body(qi_ref, di_ref, qs_ref, ds_ref, wq_ref, bq_ref, wd_ref, bd_ref,
             qo_ref, do_ref):
    def tower(i_ref, s_ref, w_ref, b_ref, o_ref):
        cnt = jnp.sum((i_ref[...] != 0).astype(jnp.float32), axis=1,
                      keepdims=True)
        avg = s_ref[...] / jnp.maximum(cnt, 1e-9)
        out = jax.lax.dot_general(
            avg, w_ref[...], (((1,), (0,)), ((), ())),
            precision=jax.lax.Precision.HIGHEST,
            preferred_element_type=jnp.float32) + b_ref[...]
        n = jnp.sqrt(jnp.sum(out * out, axis=1, keepdims=True))
        o_ref[...] = out / jnp.maximum(n, 1e-12)

    tower(qi_ref, qs_ref, wq_ref, bq_ref, qo_ref)
    tower(di_ref, ds_ref, wd_ref, bd_ref, do_ref)


def _tc_fc(qi, di, q_sum, d_sum, wqt, bq2, wdt, bd2):
    B, L = qi.shape
    blk = 2048
    grid = (B // blk,)
    row_spec = pl.BlockSpec((blk, L), lambda i: (i, 0))
    sum_spec = pl.BlockSpec((blk, EMB), lambda i: (i, 0))
    w_spec = pl.BlockSpec((EMB, EMB), lambda i: (0, 0))
    b_spec = pl.BlockSpec((1, EMB), lambda i: (0, 0))
    return pl.pallas_call(
        _tc_body,
        grid=grid,
        in_specs=[row_spec, row_spec, sum_spec, sum_spec,
                  w_spec, b_spec, w_spec, b_spec],
        out_specs=[sum_spec, sum_spec],
        out_shape=[jax.ShapeDtypeStruct((B, EMB), jnp.float32),
                   jax.ShapeDtypeStruct((B, EMB), jnp.float32)],
    )(qi, di, q_sum, d_sum, wqt, bq2, wdt, bd2)


def kernel(query_idxs, doc_idxs, query_table, doc_table, Wq, bq, Wd, bd):
    B, L = query_idxs.shape
    qi = query_idxs.astype(jnp.int32)
    di = doc_idxs.astype(jnp.int32)
    q_idx3 = qi.T.reshape(L, B // CH, CH)
    d_idx3 = di.T.reshape(L, B // CH, CH)
    q_sum, d_sum = _sc_sum(q_idx3, d_idx3, query_table, doc_table)
    q_norm, d_norm = _tc_fc(qi, di, q_sum, d_sum,
                            Wq.T, bq.reshape(1, -1), Wd.T, bd.reshape(1, -1))
    return (q_norm, d_norm)


# speedup vs baseline: 2.9499x; 1.1626x over previous
import functools

import jax
import jax.numpy as jnp
from jax import lax
from jax.experimental import pallas as pl
from jax.experimental.pallas import tpu as pltpu
from jax.experimental.pallas import tpu_sc as plsc

EMB = 64
NC, NS = 2, 16
NW = NC * NS
CH = 128


def _sc_body(L, nch, q_idx3, d_idx3, q_table, d_table, q_out, d_out,
             q_idx_v, d_idx_v, q_acc, d_acc, q_sem, d_sem):
    wid = lax.axis_index("s") * NC + lax.axis_index("c")
    bpw = nch * CH
    base_ch = wid * nch

    pltpu.sync_copy(q_idx3.at[:, pl.ds(base_ch, nch), :], q_idx_v)
    pltpu.sync_copy(d_idx3.at[:, pl.ds(base_ch, nch), :], d_idx_v)

    def fire(table, idx_v, acc, sem, l, add):
        for c in range(nch):
            pltpu.async_copy(
                table.at[idx_v.at[l, c]],
                acc.at[pl.ds(c * CH, CH), :], sem, add=add)

    def drain(table, idx_v, acc, sem):
        for c in range(nch):
            pltpu.make_async_copy(
                table.at[idx_v.at[0, c]],
                acc.at[pl.ds(c * CH, CH), :], sem).wait()

    fire(q_table, q_idx_v, q_acc, q_sem, 0, False)
    fire(d_table, d_idx_v, d_acc, d_sem, 0, False)

    def step(l, _):
        drain(q_table, q_idx_v, q_acc, q_sem)
        fire(q_table, q_idx_v, q_acc, q_sem, l, True)
        drain(d_table, d_idx_v, d_acc, d_sem)
        fire(d_table, d_idx_v, d_acc, d_sem, l, True)
        return 0

    lax.fori_loop(1, L, step, 0)
    drain(q_table, q_idx_v, q_acc, q_sem)
    drain(d_table, d_idx_v, d_acc, d_sem)
    pltpu.sync_copy(q_acc, q_out.at[pl.ds(wid * bpw, bpw), :])
    pltpu.sync_copy(d_acc, d_out.at[pl.ds(wid * bpw, bpw), :])


def _sc_sum(q_idx3, d_idx3, q_table, d_table):
    L, nblk, _ = q_idx3.shape
    B = nblk * CH
    nch = nblk // NW
    mesh = plsc.VectorSubcoreMesh(
        core_axis_name="c", subcore_axis_name="s",
        num_cores=NC, num_subcores=NS)
    f = functools.partial(
        pl.kernel,
        functools.partial(_sc_body, L, nch),
        out_type=(jax.ShapeDtypeStruct((B, EMB), jnp.float32),
                  jax.ShapeDtypeStruct((B, EMB), jnp.float32)),
        mesh=mesh,
        compiler_params=pltpu.CompilerParams(use_tc_tiling_on_sc=False),
        scratch_types=[
            pltpu.VMEM((L, nch, CH), jnp.int32),
            pltpu.VMEM((L, nch, CH), jnp.int32),
            pltpu.VMEM((nch * CH, EMB), jnp.float32),
            pltpu.VMEM((nch * CH, EMB), jnp.float32),
            pltpu.SemaphoreType.DMA,
            pltpu.SemaphoreType.DMA,
        ],
    )()
    return f(q_idx3, d_idx3, q_table, d_table)


def _tc_body(qi_ref, di_ref, qs_ref, ds_ref, wq_ref, bq_ref, wd_ref, bd_ref,
             qo_ref, do_ref):
    def tower(i_ref, s_ref, w_ref, b_ref, o_ref):
        cnt = jnp.sum((i_ref[...] != 0).astype(jnp.float32), axis=1,
                      keepdims=True)
        avg = s_ref[...] / jnp.maximum(cnt, 1e-9)
        out = jax.lax.dot_general(
            avg, w_ref[...], (((1,), (0,)), ((), ())),
            precision=jax.lax.Precision.HIGHEST,
            preferred_element_type=jnp.float32) + b_ref[...]
        n = jnp.sqrt(jnp.sum(out * out, axis=1, keepdims=True))
        o_ref[...] = out / jnp.maximum(n, 1e-12)

    tower(qi_ref, qs_ref, wq_ref, bq_ref, qo_ref)
    tower(di_ref, ds_ref, wd_ref, bd_ref, do_ref)


def _tc_fc(qi, di, q_sum, d_sum, wqt, bq2, wdt, bd2):
    B, L = qi.shape
    blk = 2048
    grid = (B // blk,)
    row_spec = pl.BlockSpec((blk, L), lambda i: (i, 0))
    sum_spec = pl.BlockSpec((blk, EMB), lambda i: (i, 0))
    w_spec = pl.BlockSpec((EMB, EMB), lambda i: (0, 0))
    b_spec = pl.BlockSpec((1, EMB), lambda i: (0, 0))
    return pl.pallas_call(
        _tc_body,
        grid=grid,
        in_specs=[row_spec, row_spec, sum_spec, sum_spec,
                  w_spec, b_spec, w_spec, b_spec],
        out_specs=[sum_spec, sum_spec],
        out_shape=[jax.ShapeDtypeStruct((B, EMB), jnp.float32),
                   jax.ShapeDtypeStruct((B, EMB), jnp.float32)],
    )(qi, di, q_sum, d_sum, wqt, bq2, wdt, bd2)


def kernel(query_idxs, doc_idxs, query_table, doc_table, Wq, bq, Wd, bd):
    B, L = query_idxs.shape
    qi = query_idxs.astype(jnp.int32)
    di = doc_idxs.astype(jnp.int32)
    q_idx3 = qi.T.reshape(L, B // CH, CH)
    d_idx3 = di.T.reshape(L, B // CH, CH)
    q_sum, d_sum = _sc_sum(q_idx3, d_idx3, query_table, doc_table)
    q_norm, d_norm = _tc_fc(qi, di, q_sum, d_sum,
                            Wq.T, bq.reshape(1, -1), Wd.T, bd.reshape(1, -1))
    return (q_norm, d_norm)
